# Initial kernel scaffold; baseline (speedup 1.0000x reference)
import functools
import jax
import jax.numpy as jnp
from jax import lax
from jax.experimental import pallas as pl
from jax.experimental.pallas import tpu as pltpu
from jax.experimental.pallas import tpu_sc as plsc

N_NODES = 10000
N_EDGES = 160000
DIM = 256
HALF = 128
N_GRAPHS = 16

NC, NS, LANES = 2, 16, 16
CHUNK = 128
N_CHUNKS = 1280
E_PAD = N_CHUNKS * CHUNK
CPT = N_CHUNKS // NS
CPW = N_CHUNKS // (NC * NS)
NPAD = 10016
RPT = N_NODES // NS
RPT_PAD = NPAD // NS

ROW_BLK = 1000
N_RB = N_NODES // ROW_BLK

_sc_mesh = plsc.VectorSubcoreMesh(core_axis_name="c", subcore_axis_name="s")


@functools.partial(
    pl.kernel,
    out_type=jax.ShapeDtypeStruct((NC, NPAD, LANES), jnp.float32),
    mesh=_sc_mesh,
    scratch_types=[
        pltpu.VMEM((CPW, CHUNK), jnp.int32),
        pltpu.VMEM((CHUNK, LANES), jnp.float32),
        pltpu.VMEM_SHARED((NPAD, LANES), jnp.float32),
    ],
)
def _deg_kernel(dst_hbm, ones_hbm, zeros_hbm, deg_hbm, idx_v, ones_v, acc_sh):
    c = lax.axis_index("c")
    s = lax.axis_index("s")
    wid = s * NC + c
    pltpu.sync_copy(zeros_hbm.at[pl.ds(s * RPT_PAD, RPT_PAD)],
                    acc_sh.at[pl.ds(s * RPT_PAD, RPT_PAD)])
    pltpu.sync_copy(ones_hbm, ones_v)
    pltpu.sync_copy(dst_hbm.at[pl.ds(wid * CPW, CPW)], idx_v)
    plsc.subcore_barrier()

    @pl.loop(0, CPW)
    def _count(j):
        pltpu.sync_copy(ones_v, acc_sh.at[idx_v.at[j]], add=True)

    plsc.subcore_barrier()
    pltpu.sync_copy(acc_sh.at[pl.ds(s * RPT_PAD, RPT_PAD)],
                    deg_hbm.at[c, pl.ds(s * RPT_PAD, RPT_PAD)])


@functools.partial(
    pl.kernel,
    out_type=jax.ShapeDtypeStruct((NC, N_NODES, HALF), jnp.float32),
    mesh=_sc_mesh,
    scratch_types=[
        pltpu.VMEM((CPT, CHUNK), jnp.int32),
        pltpu.VMEM((CPT, CHUNK), jnp.int32),
        pltpu.VMEM((CHUNK, HALF), jnp.float32),
        pltpu.VMEM_SHARED((NPAD, HALF), jnp.float32),
        pltpu.SemaphoreType.DMA,
    ],
)
def _msg_kernel(y_hbm, src_hbm, dst_hbm, out_hbm, src_v, dst_v, rows_v, acc_sh,
                sem):
    c = lax.axis_index("c")
    s = lax.axis_index("s")
    pltpu.sync_copy(src_hbm.at[c, pl.ds(s * CPT, CPT)], src_v)
    pltpu.sync_copy(dst_hbm.at[pl.ds(s * CPT, CPT)], dst_v)
    pltpu.sync_copy(y_hbm.at[pl.ds(c * N_NODES + s * RPT, RPT)],
                    acc_sh.at[pl.ds(s * RPT, RPT)])
    plsc.subcore_barrier()

    @pl.loop(0, CPT)
    def _edges(j):
        pltpu.async_copy(y_hbm.at[src_v.at[j]], rows_v, sem).wait()
        pltpu.sync_copy(rows_v, acc_sh.at[dst_v.at[j]], add=True)

    plsc.subcore_barrier()
    pltpu.sync_copy(acc_sh.at[pl.ds(s * RPT, RPT)],
                    out_hbm.at[c, pl.ds(s * RPT, RPT)])


def _dinv_body(deg_ref, dinv_ref):
    d = deg_ref[0, :, 0:1] + deg_ref[1, :, 0:1] + 1.0
    dinv_ref[...] = lax.rsqrt(d)


def _dinv_call(deg):
    return pl.pallas_call(
        _dinv_body,
        grid=(N_RB,),
        in_specs=[pl.BlockSpec((2, ROW_BLK, LANES), lambda i: (0, i, 0))],
        out_specs=pl.BlockSpec((ROW_BLK, 1), lambda i: (i, 0)),
        out_shape=jax.ShapeDtypeStruct((N_NODES, 1), jnp.float32),
    )(deg)


def _mm1_body(x_ref, w_ref, dinv_ref, y_ref):
    acc = jnp.dot(x_ref[...], w_ref[...], preferred_element_type=jnp.float32)
    y_ref[...] = acc * dinv_ref[...]


def _mm1_call(x, w, dinv):
    return pl.pallas_call(
        _mm1_body,
        grid=(N_RB, NC),
        in_specs=[
            pl.BlockSpec((ROW_BLK, DIM), lambda i, j: (i, 0)),
            pl.BlockSpec((DIM, HALF), lambda i, j: (0, j)),
            pl.BlockSpec((ROW_BLK, 1), lambda i, j: (i, 0)),
        ],
        out_specs=pl.BlockSpec((ROW_BLK, HALF), lambda i, j: (j * N_RB + i, 0)),
        out_shape=jax.ShapeDtypeStruct((NC * N_NODES, HALF), jnp.float32),
    )(x, w, dinv)


def _lin_body(acca_ref, accb_ref, dinv_ref, b_ref, w_ref, y_ref):
    h = jnp.concatenate([acca_ref[...], accb_ref[...]], axis=1)
    h = jnp.maximum(h * dinv_ref[...] + b_ref[...], 0.0)
    y_ref[...] = (
        jnp.dot(h, w_ref[...], preferred_element_type=jnp.float32)
        * dinv_ref[...]
    )


def _lin_call(accf, dinv, b, w):
    return pl.pallas_call(
        _lin_body,
        grid=(N_RB, NC),
        in_specs=[
            pl.BlockSpec((ROW_BLK, HALF), lambda i, j: (i, 0)),
            pl.BlockSpec((ROW_BLK, HALF), lambda i, j: (N_RB + i, 0)),
            pl.BlockSpec((ROW_BLK, 1), lambda i, j: (i, 0)),
            pl.BlockSpec((1, DIM), lambda i, j: (0, 0)),
            pl.BlockSpec((DIM, HALF), lambda i, j: (0, j)),
        ],
        out_specs=pl.BlockSpec((ROW_BLK, HALF), lambda i, j: (j * N_RB + i, 0)),
        out_shape=jax.ShapeDtypeStruct((NC * N_NODES, HALF), jnp.float32),
    )(accf, accf, dinv, b, w)


def _pool_body(acca_ref, accb_ref, dinv_ref, b_ref, batch_ref, out_ref, cnts):
    i = pl.program_id(0)

    @pl.when(i == 0)
    def _():
        out_ref[...] = jnp.zeros_like(out_ref)
        cnts[...] = jnp.zeros_like(cnts)

    h = jnp.concatenate([acca_ref[...], accb_ref[...]], axis=1)
    h = h * dinv_ref[...] + b_ref[...]
    onehot = (batch_ref[...] ==
              lax.broadcasted_iota(jnp.int32, (1, N_GRAPHS), 1)
              ).astype(jnp.float32)
    out_ref[...] += lax.dot_general(
        onehot, h, (((0,), (0,)), ((), ())), preferred_element_type=jnp.float32)
    cnt = jnp.sum(onehot, axis=0)[:, None]
    cnts[...] += jnp.broadcast_to(cnt, (N_GRAPHS, DIM))

    @pl.when(i == pl.num_programs(0) - 1)
    def _():
        out_ref[...] = out_ref[...] / jnp.maximum(cnts[...], 1.0)


def _pool_call(accf, dinv, b, batch2):
    return pl.pallas_call(
        _pool_body,
        grid=(N_RB,),
        in_specs=[
            pl.BlockSpec((ROW_BLK, HALF), lambda i: (i, 0)),
            pl.BlockSpec((ROW_BLK, HALF), lambda i: (N_RB + i, 0)),
            pl.BlockSpec((ROW_BLK, 1), lambda i: (i, 0)),
            pl.BlockSpec((1, DIM), lambda i: (0, 0)),
            pl.BlockSpec((ROW_BLK, 1), lambda i: (i, 0)),
        ],
        out_specs=pl.BlockSpec((N_GRAPHS, DIM), lambda i: (0, 0)),
        out_shape=jax.ShapeDtypeStruct((N_GRAPHS, DIM), jnp.float32),
        scratch_shapes=[pltpu.VMEM((N_GRAPHS, DIM), jnp.float32)],
    )(accf, accf, dinv, b, batch2)


@jax.jit
def kernel(x, edge_index, ptr, batch, W1, b1, W2, b2, W3, b3):
    x = x.astype(jnp.float32)
    src = edge_index[0]
    dst = edge_index[1]
    pad = E_PAD - N_EDGES
    srcp = jnp.concatenate([src, jnp.zeros((pad,), jnp.int32)])
    dstp = jnp.concatenate([dst, jnp.full((pad,), N_NODES, jnp.int32)])
    src2 = jnp.stack([srcp, srcp + N_NODES]).reshape(NC, N_CHUNKS, CHUNK)
    dst2 = dstp.reshape(N_CHUNKS, CHUNK)
    ones_rows = jnp.ones((CHUNK, LANES), jnp.float32)
    zeros_init = jnp.zeros((NPAD, LANES), jnp.float32)
    batch2 = batch.reshape(N_NODES, 1)

    deg = _deg_kernel(dst2, ones_rows, zeros_init)
    dinv = _dinv_call(deg)

    y = _mm1_call(x, W1, dinv)
    acc = _msg_kernel(y, src2, dst2).reshape(NC * N_NODES, HALF)
    y = _lin_call(acc, dinv, b1.reshape(1, DIM), W2)
    acc = _msg_kernel(y, src2, dst2).reshape(NC * N_NODES, HALF)
    y = _lin_call(acc, dinv, b2.reshape(1, DIM), W3)
    acc = _msg_kernel(y, src2, dst2).reshape(NC * N_NODES, HALF)
    out = _pool_call(acc, dinv, b3.reshape(1, DIM), batch2)
    return out.reshape(-1)


# speedup vs baseline: 5.9364x; 5.9364x over previous
import functools
import jax
import jax.numpy as jnp
from jax import lax
from jax.experimental import pallas as pl
from jax.experimental.pallas import tpu as pltpu
from jax.experimental.pallas import tpu_sc as plsc

N_NODES = 10000
N_EDGES = 160000
DIM = 256
HALF = 128
N_GRAPHS = 16

NC, NS, LANES = 2, 16, 16
CHUNK = 128
N_CHUNKS = 1280
E_PAD = N_CHUNKS * CHUNK
CPT = N_CHUNKS // NS
CPW = N_CHUNKS // (NC * NS)
NPAD = 10112
RPT = NPAD // NS

ROW_BLK = 1000
N_RB = N_NODES // ROW_BLK

def _deg_body(dst_hbm, ones_hbm, zeros_hbm, deg_hbm, idx_v, ones_v, acc_sh):
    c = lax.axis_index("c")
    s = lax.axis_index("s")
    wid = s * NC + c
    pltpu.sync_copy(zeros_hbm.at[pl.ds(s * RPT, RPT)],
                    acc_sh.at[pl.ds(s * RPT, RPT)])
    pltpu.sync_copy(ones_hbm, ones_v)
    pltpu.sync_copy(dst_hbm.at[pl.ds(wid * CPW, CPW)], idx_v)
    plsc.subcore_barrier()

    @pl.loop(0, CPW)
    def _count(j):
        pltpu.sync_copy(ones_v, acc_sh.at[idx_v.at[j]], add=True)

    plsc.subcore_barrier()
    pltpu.sync_copy(acc_sh.at[pl.ds(s * RPT, RPT)],
                    deg_hbm.at[c, pl.ds(s * RPT, RPT)])


def _msg_body(y_hbm, src_hbm, dst_hbm, out_hbm, src_v, dst_v, rows_v, acc_sh,
              sem):
    c = lax.axis_index("c")
    s = lax.axis_index("s")
    pltpu.sync_copy(src_hbm.at[c, pl.ds(s * CPT, CPT)], src_v)
    pltpu.sync_copy(dst_hbm.at[pl.ds(s * CPT, CPT)], dst_v)
    pltpu.sync_copy(y_hbm.at[pl.ds(c * NPAD + s * RPT, RPT)],
                    acc_sh.at[pl.ds(s * RPT, RPT)])
    plsc.subcore_barrier()

    @pl.loop(0, CPT)
    def _edges(j):
        pltpu.async_copy(y_hbm.at[src_v.at[j]], rows_v, sem).wait()
        pltpu.sync_copy(rows_v, acc_sh.at[dst_v.at[j]], add=True)

    plsc.subcore_barrier()
    pltpu.sync_copy(acc_sh.at[pl.ds(s * RPT, RPT)],
                    out_hbm.at[c, pl.ds(s * RPT, RPT)])


@functools.cache
def _sc_kernels():
    mesh = plsc.VectorSubcoreMesh(
        core_axis_name="c", subcore_axis_name="s",
        num_cores=NC, num_subcores=NS)
    deg_kernel = pl.kernel(
        _deg_body,
        out_type=jax.ShapeDtypeStruct((NC, NPAD, HALF), jnp.float32),
        mesh=mesh,
        scratch_types=[
            pltpu.VMEM((CPW, CHUNK), jnp.int32),
            pltpu.VMEM((CHUNK, HALF), jnp.float32),
            pltpu.VMEM_SHARED((NPAD, HALF), jnp.float32),
        ],
    )
    msg_kernel = pl.kernel(
        _msg_body,
        out_type=jax.ShapeDtypeStruct((NC, NPAD, HALF), jnp.float32),
        mesh=mesh,
        scratch_types=[
            pltpu.VMEM((CPT, CHUNK), jnp.int32),
            pltpu.VMEM((CPT, CHUNK), jnp.int32),
            pltpu.VMEM((CHUNK, HALF), jnp.float32),
            pltpu.VMEM_SHARED((NPAD, HALF), jnp.float32),
            pltpu.SemaphoreType.DMA,
        ],
    )
    return deg_kernel, msg_kernel


def _dinv_body(deg_ref, dinv_ref):
    d = deg_ref[0, :, 0:1] + deg_ref[1, :, 0:1] + 1.0
    dinv_ref[...] = lax.rsqrt(d)


def _dinv_call(deg):
    return pl.pallas_call(
        _dinv_body,
        grid=(N_RB,),
        in_specs=[pl.BlockSpec((2, ROW_BLK, HALF), lambda i: (0, i, 0))],
        out_specs=pl.BlockSpec((ROW_BLK, 1), lambda i: (i, 0)),
        out_shape=jax.ShapeDtypeStruct((N_NODES, 1), jnp.float32),
    )(deg)


def _mm1_body(x_ref, w_ref, dinv_ref, y_ref):
    acc = jnp.dot(x_ref[...], w_ref[...], preferred_element_type=jnp.float32)
    y_ref[0] = acc * dinv_ref[...]


def _mm1_call(x, w, dinv):
    return pl.pallas_call(
        _mm1_body,
        grid=(N_RB, NC),
        in_specs=[
            pl.BlockSpec((ROW_BLK, DIM), lambda i, j: (i, 0)),
            pl.BlockSpec((DIM, HALF), lambda i, j: (0, j)),
            pl.BlockSpec((ROW_BLK, 1), lambda i, j: (i, 0)),
        ],
        out_specs=pl.BlockSpec((1, ROW_BLK, HALF), lambda i, j: (j, i, 0)),
        out_shape=jax.ShapeDtypeStruct((NC, NPAD, HALF), jnp.float32),
    )(x, w, dinv)


def _lin_body(acca_ref, accb_ref, dinv_ref, b_ref, w_ref, y_ref):
    h = jnp.concatenate([acca_ref[0], accb_ref[0]], axis=1)
    h = jnp.maximum(h * dinv_ref[...] + b_ref[...], 0.0)
    y_ref[0] = (
        jnp.dot(h, w_ref[...], preferred_element_type=jnp.float32)
        * dinv_ref[...]
    )


def _lin_call(accf, dinv, b, w):
    return pl.pallas_call(
        _lin_body,
        grid=(N_RB, NC),
        in_specs=[
            pl.BlockSpec((1, ROW_BLK, HALF), lambda i, j: (0, i, 0)),
            pl.BlockSpec((1, ROW_BLK, HALF), lambda i, j: (1, i, 0)),
            pl.BlockSpec((ROW_BLK, 1), lambda i, j: (i, 0)),
            pl.BlockSpec((1, DIM), lambda i, j: (0, 0)),
            pl.BlockSpec((DIM, HALF), lambda i, j: (0, j)),
        ],
        out_specs=pl.BlockSpec((1, ROW_BLK, HALF), lambda i, j: (j, i, 0)),
        out_shape=jax.ShapeDtypeStruct((NC, NPAD, HALF), jnp.float32),
    )(accf, accf, dinv, b, w)


def _pool_body(acca_ref, accb_ref, dinv_ref, b_ref, batch_ref, out_ref, cnts):
    i = pl.program_id(0)

    @pl.when(i == 0)
    def _():
        out_ref[...] = jnp.zeros_like(out_ref)
        cnts[...] = jnp.zeros_like(cnts)

    h = jnp.concatenate([acca_ref[0], accb_ref[0]], axis=1)
    h = h * dinv_ref[...] + b_ref[...]
    onehot = (batch_ref[...] ==
              lax.broadcasted_iota(jnp.int32, (1, N_GRAPHS), 1)
              ).astype(jnp.float32)
    out_ref[...] += lax.dot_general(
        onehot, h, (((0,), (0,)), ((), ())), preferred_element_type=jnp.float32)
    cnt = jnp.sum(onehot, axis=0)[:, None]
    cnts[...] += jnp.broadcast_to(cnt, (N_GRAPHS, DIM))

    @pl.when(i == pl.num_programs(0) - 1)
    def _():
        out_ref[...] = out_ref[...] / jnp.maximum(cnts[...], 1.0)


def _pool_call(accf, dinv, b, batch2):
    return pl.pallas_call(
        _pool_body,
        grid=(N_RB,),
        in_specs=[
            pl.BlockSpec((1, ROW_BLK, HALF), lambda i: (0, i, 0)),
            pl.BlockSpec((1, ROW_BLK, HALF), lambda i: (1, i, 0)),
            pl.BlockSpec((ROW_BLK, 1), lambda i: (i, 0)),
            pl.BlockSpec((1, DIM), lambda i: (0, 0)),
            pl.BlockSpec((ROW_BLK, 1), lambda i: (i, 0)),
        ],
        out_specs=pl.BlockSpec((N_GRAPHS, DIM), lambda i: (0, 0)),
        out_shape=jax.ShapeDtypeStruct((N_GRAPHS, DIM), jnp.float32),
        scratch_shapes=[pltpu.VMEM((N_GRAPHS, DIM), jnp.float32)],
    )(accf, accf, dinv, b, batch2)


@jax.jit
def kernel(x, edge_index, ptr, batch, W1, b1, W2, b2, W3, b3):
    x = x.astype(jnp.float32)
    src = edge_index[0]
    dst = edge_index[1]
    pad = E_PAD - N_EDGES
    srcp = jnp.concatenate([src, jnp.zeros((pad,), jnp.int32)])
    dstp = jnp.concatenate([dst, jnp.full((pad,), N_NODES, jnp.int32)])
    src2 = jnp.stack([srcp, srcp + NPAD]).reshape(NC, N_CHUNKS, CHUNK)
    dst2 = dstp.reshape(N_CHUNKS, CHUNK)
    ones_rows = jnp.ones((CHUNK, HALF), jnp.float32)
    zeros_init = jnp.zeros((NPAD, HALF), jnp.float32)
    batch2 = batch.reshape(N_NODES, 1)

    deg_kernel, msg_kernel = _sc_kernels()
    deg = deg_kernel(dst2, ones_rows, zeros_init)
    dinv = _dinv_call(deg)

    y = _mm1_call(x, W1, dinv)
    acc = msg_kernel(y.reshape(NC * NPAD, HALF), src2, dst2)
    y = _lin_call(acc, dinv, b1.reshape(1, DIM), W2)
    acc = msg_kernel(y.reshape(NC * NPAD, HALF), src2, dst2)
    y = _lin_call(acc, dinv, b2.reshape(1, DIM), W3)
    acc = msg_kernel(y.reshape(NC * NPAD, HALF), src2, dst2)
    out = _pool_call(acc, dinv, b3.reshape(1, DIM), batch2)
    return out.reshape(-1)


# speedup vs baseline: 6.4885x; 1.0930x over previous
import functools
import jax
import jax.numpy as jnp
from jax import lax
from jax.experimental import pallas as pl
from jax.experimental.pallas import tpu as pltpu
from jax.experimental.pallas import tpu_sc as plsc

N_NODES = 10000
N_EDGES = 160000
DIM = 256
HALF = 128
N_GRAPHS = 16

NC, NS, LANES = 2, 16, 16
CHUNK = 128
N_CHUNKS = 1280
E_PAD = N_CHUNKS * CHUNK
CPT = N_CHUNKS // NS
CPW = N_CHUNKS // (NC * NS)
NPAD = 10112
RPT = NPAD // NS
NBUF = 2
HCPT = CPT // 2

ROW_BLK = 1000
N_RB = N_NODES // ROW_BLK

def _deg_body(dst_hbm, ones_hbm, zeros_hbm, deg_hbm, idx_v, ones_v, acc_sh):
    c = lax.axis_index("c")
    s = lax.axis_index("s")
    wid = s * NC + c
    pltpu.sync_copy(zeros_hbm.at[pl.ds(s * RPT, RPT)],
                    acc_sh.at[pl.ds(s * RPT, RPT)])
    pltpu.sync_copy(ones_hbm, ones_v)
    pltpu.sync_copy(dst_hbm.at[pl.ds(wid * CPW, CPW)], idx_v)
    plsc.subcore_barrier()

    @pl.loop(0, CPW)
    def _count(j):
        pltpu.sync_copy(ones_v, acc_sh.at[idx_v.at[j]], add=True)

    plsc.subcore_barrier()
    pltpu.sync_copy(acc_sh.at[pl.ds(s * RPT, RPT)],
                    deg_hbm.at[c, pl.ds(s * RPT, RPT)])


def _msg_body(y_hbm, src_hbm, dst_hbm, out_hbm, src_v, dst_v, bufs, acc_sh,
              gsems, ssems):
    c = lax.axis_index("c")
    s = lax.axis_index("s")
    pltpu.sync_copy(y_hbm.at[pl.ds(c * NPAD + s * RPT, RPT)],
                    acc_sh.at[pl.ds(s * RPT, RPT)])
    plsc.subcore_barrier()

    for p in range(2):
        base = p * HCPT
        pltpu.sync_copy(src_hbm.at[c, pl.ds(s * CPT + base, HCPT)], src_v)
        pltpu.sync_copy(dst_hbm.at[pl.ds(s * CPT + base, HCPT)], dst_v)
        for b in range(NBUF):
            pltpu.async_copy(y_hbm.at[src_v.at[b]], bufs[b], gsems[b])

        @pl.loop(0, HCPT, step=NBUF)
        def _edges(j):
            for b in range(NBUF):
                pltpu.make_async_copy(
                    y_hbm.at[src_v.at[j + b]], bufs[b], gsems[b]).wait()
                pltpu.async_copy(
                    bufs[b], acc_sh.at[dst_v.at[j + b]], ssems[b], add=True)
            for b in range(NBUF):
                pltpu.make_async_copy(
                    bufs[b], acc_sh.at[dst_v.at[j + b]], ssems[b]).wait()

                @pl.when(j + NBUF < HCPT)
                def _():
                    pltpu.async_copy(
                        y_hbm.at[src_v.at[j + b + NBUF]], bufs[b], gsems[b])

    plsc.subcore_barrier()
    pltpu.sync_copy(acc_sh.at[pl.ds(s * RPT, RPT)],
                    out_hbm.at[c, pl.ds(s * RPT, RPT)])


@functools.cache
def _sc_kernels():
    mesh = plsc.VectorSubcoreMesh(
        core_axis_name="c", subcore_axis_name="s",
        num_cores=NC, num_subcores=NS)
    deg_kernel = pl.kernel(
        _deg_body,
        out_type=jax.ShapeDtypeStruct((NC, NPAD, HALF), jnp.float32),
        mesh=mesh,
        scratch_types=[
            pltpu.VMEM((CPW, CHUNK), jnp.int32),
            pltpu.VMEM((CHUNK, HALF), jnp.float32),
            pltpu.VMEM_SHARED((NPAD, HALF), jnp.float32),
        ],
    )
    msg_kernel = pl.kernel(
        _msg_body,
        out_type=jax.ShapeDtypeStruct((NC, NPAD, HALF), jnp.float32),
        mesh=mesh,
        scratch_types=[
            pltpu.VMEM((HCPT, CHUNK), jnp.int32),
            pltpu.VMEM((HCPT, CHUNK), jnp.int32),
            [pltpu.VMEM((CHUNK, HALF), jnp.float32)] * NBUF,
            pltpu.VMEM_SHARED((NPAD, HALF), jnp.float32),
            [pltpu.SemaphoreType.DMA] * NBUF,
            [pltpu.SemaphoreType.DMA] * NBUF,
        ],
    )
    return deg_kernel, msg_kernel


def _dinv_body(deg_ref, dinv_ref):
    d = deg_ref[0, :, 0:1] + deg_ref[1, :, 0:1] + 1.0
    dinv_ref[...] = lax.rsqrt(d)


def _dinv_call(deg):
    return pl.pallas_call(
        _dinv_body,
        grid=(N_RB,),
        in_specs=[pl.BlockSpec((2, ROW_BLK, HALF), lambda i: (0, i, 0))],
        out_specs=pl.BlockSpec((ROW_BLK, 1), lambda i: (i, 0)),
        out_shape=jax.ShapeDtypeStruct((N_NODES, 1), jnp.float32),
    )(deg)


def _mm1_body(x_ref, w_ref, dinv_ref, y_ref):
    acc = jnp.dot(x_ref[...], w_ref[...], preferred_element_type=jnp.float32)
    y_ref[0] = acc * dinv_ref[...]


def _mm1_call(x, w, dinv):
    return pl.pallas_call(
        _mm1_body,
        grid=(N_RB, NC),
        in_specs=[
            pl.BlockSpec((ROW_BLK, DIM), lambda i, j: (i, 0)),
            pl.BlockSpec((DIM, HALF), lambda i, j: (0, j)),
            pl.BlockSpec((ROW_BLK, 1), lambda i, j: (i, 0)),
        ],
        out_specs=pl.BlockSpec((1, ROW_BLK, HALF), lambda i, j: (j, i, 0)),
        out_shape=jax.ShapeDtypeStruct((NC, NPAD, HALF), jnp.float32),
    )(x, w, dinv)


def _lin_body(acca_ref, accb_ref, dinv_ref, b_ref, w_ref, y_ref):
    h = jnp.concatenate([acca_ref[0], accb_ref[0]], axis=1)
    h = jnp.maximum(h * dinv_ref[...] + b_ref[...], 0.0)
    y_ref[0] = (
        jnp.dot(h, w_ref[...], preferred_element_type=jnp.float32)
        * dinv_ref[...]
    )


def _lin_call(accf, dinv, b, w):
    return pl.pallas_call(
        _lin_body,
        grid=(N_RB, NC),
        in_specs=[
            pl.BlockSpec((1, ROW_BLK, HALF), lambda i, j: (0, i, 0)),
            pl.BlockSpec((1, ROW_BLK, HALF), lambda i, j: (1, i, 0)),
            pl.BlockSpec((ROW_BLK, 1), lambda i, j: (i, 0)),
            pl.BlockSpec((1, DIM), lambda i, j: (0, 0)),
            pl.BlockSpec((DIM, HALF), lambda i, j: (0, j)),
        ],
        out_specs=pl.BlockSpec((1, ROW_BLK, HALF), lambda i, j: (j, i, 0)),
        out_shape=jax.ShapeDtypeStruct((NC, NPAD, HALF), jnp.float32),
    )(accf, accf, dinv, b, w)


def _pool_body(acca_ref, accb_ref, dinv_ref, b_ref, batch_ref, out_ref, cnts):
    i = pl.program_id(0)

    @pl.when(i == 0)
    def _():
        out_ref[...] = jnp.zeros_like(out_ref)
        cnts[...] = jnp.zeros_like(cnts)

    h = jnp.concatenate([acca_ref[0], accb_ref[0]], axis=1)
    h = h * dinv_ref[...] + b_ref[...]
    onehot = (batch_ref[...] ==
              lax.broadcasted_iota(jnp.int32, (1, N_GRAPHS), 1)
              ).astype(jnp.float32)
    out_ref[...] += lax.dot_general(
        onehot, h, (((0,), (0,)), ((), ())), preferred_element_type=jnp.float32)
    cnt = jnp.sum(onehot, axis=0)[:, None]
    cnts[...] += jnp.broadcast_to(cnt, (N_GRAPHS, DIM))

    @pl.when(i == pl.num_programs(0) - 1)
    def _():
        out_ref[...] = out_ref[...] / jnp.maximum(cnts[...], 1.0)


def _pool_call(accf, dinv, b, batch2):
    return pl.pallas_call(
        _pool_body,
        grid=(N_RB,),
        in_specs=[
            pl.BlockSpec((1, ROW_BLK, HALF), lambda i: (0, i, 0)),
            pl.BlockSpec((1, ROW_BLK, HALF), lambda i: (1, i, 0)),
            pl.BlockSpec((ROW_BLK, 1), lambda i: (i, 0)),
            pl.BlockSpec((1, DIM), lambda i: (0, 0)),
            pl.BlockSpec((ROW_BLK, 1), lambda i: (i, 0)),
        ],
        out_specs=pl.BlockSpec((N_GRAPHS, DIM), lambda i: (0, 0)),
        out_shape=jax.ShapeDtypeStruct((N_GRAPHS, DIM), jnp.float32),
        scratch_shapes=[pltpu.VMEM((N_GRAPHS, DIM), jnp.float32)],
    )(accf, accf, dinv, b, batch2)


@jax.jit
def kernel(x, edge_index, ptr, batch, W1, b1, W2, b2, W3, b3):
    x = x.astype(jnp.float32)
    src = edge_index[0]
    dst = edge_index[1]
    pad = E_PAD - N_EDGES
    srcp = jnp.concatenate([src, jnp.zeros((pad,), jnp.int32)])
    dstp = jnp.concatenate([dst, jnp.full((pad,), N_NODES, jnp.int32)])
    src2 = jnp.stack([srcp, srcp + NPAD]).reshape(NC, N_CHUNKS, CHUNK)
    dst2 = dstp.reshape(N_CHUNKS, CHUNK)
    ones_rows = jnp.ones((CHUNK, HALF), jnp.float32)
    zeros_init = jnp.zeros((NPAD, HALF), jnp.float32)
    batch2 = batch.reshape(N_NODES, 1)

    deg_kernel, msg_kernel = _sc_kernels()
    deg = deg_kernel(dst2, ones_rows, zeros_init)
    dinv = _dinv_call(deg)

    y = _mm1_call(x, W1, dinv)
    acc = msg_kernel(y.reshape(NC * NPAD, HALF), src2, dst2)
    y = _lin_call(acc, dinv, b1.reshape(1, DIM), W2)
    acc = msg_kernel(y.reshape(NC * NPAD, HALF), src2, dst2)
    y = _lin_call(acc, dinv, b2.reshape(1, DIM), W3)
    acc = msg_kernel(y.reshape(NC * NPAD, HALF), src2, dst2)
    out = _pool_call(acc, dinv, b3.reshape(1, DIM), batch2)
    return out.reshape(-1)
